# baseline (speedup 1.0000x reference)
import functools

import jax
import jax.numpy as jnp
from jax import lax
from jax.experimental import pallas as pl
from jax.experimental.pallas import tpu as pltpu
from jax.experimental.pallas import tpu_sc as plsc

HIDDEN = 1024
FF = 2 * HIDDEN
NUM_EXPERTS = 8
N_TOKENS = 4096
TOP_K = 2
BLK = 128
NBLK = 72
S_PAD = NBLK * BLK
CHUNK = 512

NW = 32
TPW = N_TOKENS // NW


def _gelu_exact(x):
    return x * 0.5 * (1.0 + lax.erf(x * 0.7071067811865476))


def _meta_body(tokens_ref, wr_ref, slot_ref, p_ref, be_ref):
    x = tokens_ref[...]
    logits = lax.dot_general(x, wr_ref[...], (((1,), (1,)), ((), ())),
                             preferred_element_type=jnp.float32)
    idx8 = lax.broadcasted_iota(jnp.int32, logits.shape, 1)
    m1 = jnp.max(logits, axis=-1, keepdims=True)
    a1 = jnp.min(jnp.where(logits == m1, idx8, NUM_EXPERTS), axis=-1,
                 keepdims=True)
    l2 = jnp.where(idx8 == a1, jnp.float32(-1e30), logits)
    m2 = jnp.max(l2, axis=-1, keepdims=True)
    a2 = jnp.min(jnp.where(l2 == m2, idx8, NUM_EXPERTS), axis=-1,
                 keepdims=True)
    p1 = 1.0 / (1.0 + jnp.exp(m2 - m1))
    p2 = 1.0 - p1

    oh = (idx8 == a1).astype(jnp.float32) + (idx8 == a2).astype(jnp.float32)
    run = jnp.zeros((1, NUM_EXPERTS), jnp.float32)
    cum_chunks = []
    ri = lax.broadcasted_iota(jnp.int32, (CHUNK, CHUNK), 0)
    ci = lax.broadcasted_iota(jnp.int32, (CHUNK, CHUNK), 1)
    lstrict = (ci < ri).astype(jnp.float32)
    for c in range(N_TOKENS // CHUNK):
        blk = oh[c * CHUNK:(c + 1) * CHUNK]
        cum_chunks.append(
            lax.dot_general(lstrict, blk, (((1,), (0,)), ((), ())),
                            preferred_element_type=jnp.float32) + run)
        run = run + jnp.sum(blk, axis=0, keepdims=True)
    cum = jnp.concatenate(cum_chunks, axis=0)
    counts = run

    nblk_e = jnp.ceil(counts * (1.0 / BLK))
    er = lax.broadcasted_iota(jnp.int32, (NUM_EXPERTS, NUM_EXPERTS), 0)
    ec = lax.broadcasted_iota(jnp.int32, (NUM_EXPERTS, NUM_EXPERTS), 1)
    estrict = (er < ec).astype(jnp.float32)
    base_blk = lax.dot_general(nblk_e, estrict, (((1,), (0,)), ((), ())),
                               preferred_element_type=jnp.float32)
    base = base_blk * float(BLK)

    base_bc = jnp.broadcast_to(base, (N_TOKENS, NUM_EXPERTS))
    slot0 = jnp.sum(jnp.where(idx8 == a1, base_bc + cum, 0.0), axis=-1,
                    keepdims=True)
    slot1 = jnp.sum(jnp.where(idx8 == a2, base_bc + cum, 0.0), axis=-1,
                    keepdims=True)
    slot_ref[...] = jnp.concatenate([slot0, slot1], axis=1).astype(jnp.int32)
    p_ref[...] = jnp.concatenate([p1, p2], axis=1)

    bi = lax.broadcasted_iota(jnp.int32, (NBLK, NUM_EXPERTS), 0).astype(
        jnp.float32)
    cond = (jnp.broadcast_to(base_blk, (NBLK, NUM_EXPERTS)) <= bi)
    be_ref[...] = (jnp.sum(cond.astype(jnp.float32), axis=-1, keepdims=True)
                   - 1.0).astype(jnp.int32)


def _route_meta(tokens, Wr):
    return pl.pallas_call(
        _meta_body,
        in_specs=[
            pl.BlockSpec((N_TOKENS, HIDDEN), lambda: (0, 0)),
            pl.BlockSpec((NUM_EXPERTS, HIDDEN), lambda: (0, 0)),
        ],
        out_specs=[
            pl.BlockSpec((N_TOKENS, TOP_K), lambda: (0, 0)),
            pl.BlockSpec((N_TOKENS, TOP_K), lambda: (0, 0)),
            pl.BlockSpec((NBLK, 1), lambda: (0, 0)),
        ],
        out_shape=[
            jax.ShapeDtypeStruct((N_TOKENS, TOP_K), jnp.int32),
            jax.ShapeDtypeStruct((N_TOKENS, TOP_K), jnp.float32),
            jax.ShapeDtypeStruct((NBLK, 1), jnp.int32),
        ],
    )(tokens, Wr)


_NBUF = 3


def _dispatch_sc_body(tokens_hbm, slot0_hbm, slot1_hbm, disp_hbm,
                      idx0_v, idx1_v, xbuf, semin, semout):
    wid = lax.axis_index("s") * 2 + lax.axis_index("c")
    base = wid * TPW
    nch = TPW // 16
    pltpu.sync_copy(slot0_hbm.at[pl.ds(base, TPW)], idx0_v)
    pltpu.sync_copy(slot1_hbm.at[pl.ds(base, TPW)], idx1_v)
    stages = {}
    for ch in range(_NBUF):
        stages[ch] = pltpu.async_copy(
            tokens_hbm.at[pl.ds(base + ch * 16, 16)], xbuf.at[ch], semin)
    scats = []
    for ch in range(nch):
        b = ch % _NBUF
        stages[ch].wait()
        i0 = idx0_v[pl.ds(ch * 16, 16)]
        scats.append(pltpu.async_copy(xbuf.at[b], disp_hbm.at[i0], semout))
        i1 = idx1_v[pl.ds(ch * 16, 16)]
        scats.append(pltpu.async_copy(xbuf.at[b], disp_hbm.at[i1], semout))
        nxt = ch + _NBUF
        if nxt < nch:
            scats[2 * ch].wait()
            scats[2 * ch + 1].wait()
            stages[nxt] = pltpu.async_copy(
                tokens_hbm.at[pl.ds(base + nxt * 16, 16)], xbuf.at[b], semin)
    for u in range(2 * (nch - _NBUF), 2 * nch):
        scats[u].wait()


def _gmm_body(be_ref, x_ref, w1_ref, b1_ref, w2_ref, b2_ref, y_ref):
    del be_ref
    h = lax.dot_general(x_ref[...], w1_ref[0], (((1,), (1,)), ((), ())),
                        preferred_element_type=jnp.float32)
    h = _gelu_exact(h + b1_ref[0])
    y_ref[...] = lax.dot_general(h, w2_ref[0], (((1,), (1,)), ((), ())),
                                 preferred_element_type=jnp.float32) + b2_ref[0]


def _gmm(be, disp, W1, b1, W2, b2):
    grid_spec = pltpu.PrefetchScalarGridSpec(
        num_scalar_prefetch=1,
        grid=(NBLK,),
        in_specs=[
            pl.BlockSpec((BLK, HIDDEN), lambda i, be: (i, 0)),
            pl.BlockSpec((1, FF, HIDDEN), lambda i, be: (be[i], 0, 0)),
            pl.BlockSpec((1, 1, FF), lambda i, be: (be[i], 0, 0)),
            pl.BlockSpec((1, HIDDEN, FF), lambda i, be: (be[i], 0, 0)),
            pl.BlockSpec((1, 1, HIDDEN), lambda i, be: (be[i], 0, 0)),
        ],
        out_specs=pl.BlockSpec((BLK, HIDDEN), lambda i, be: (i, 0)),
    )
    return pl.pallas_call(
        _gmm_body,
        grid_spec=grid_spec,
        out_shape=jax.ShapeDtypeStruct((S_PAD, HIDDEN), jnp.float32),
    )(be, disp, W1, b1[:, None, :], W2, b2[:, None, :])


def _gather_sc_body(y_hbm, slot0_hbm, slot1_hbm, g0_hbm, g1_hbm,
                    idx0_v, idx1_v, rbuf, semin, semout):
    wid = lax.axis_index("s") * 2 + lax.axis_index("c")
    base = wid * TPW
    nch = TPW // 16
    pltpu.sync_copy(slot0_hbm.at[pl.ds(base, TPW)], idx0_v)
    pltpu.sync_copy(slot1_hbm.at[pl.ds(base, TPW)], idx1_v)

    def _idx(u):
        k, ch = u % 2, u // 2
        iv = idx0_v if k == 0 else idx1_v
        return iv[pl.ds(ch * 16, 16)]

    def _dst(u):
        k, ch = u % 2, u // 2
        g = g0_hbm if k == 0 else g1_hbm
        return g.at[pl.ds(base + ch * 16, 16)]

    nun = 2 * nch
    gath = {}
    for u in range(_NBUF):
        gath[u] = pltpu.async_copy(y_hbm.at[_idx(u)], rbuf.at[u], semin)
    outs = []
    for u in range(nun):
        b = u % _NBUF
        gath[u].wait()
        outs.append(pltpu.async_copy(rbuf.at[b], _dst(u), semout))
        nxt = u + _NBUF
        if nxt < nun:
            outs[u].wait()
            gath[nxt] = pltpu.async_copy(y_hbm.at[_idx(nxt)], rbuf.at[b],
                                         semin)
    for u in range(nun - _NBUF, nun):
        outs[u].wait()


@functools.lru_cache(maxsize=None)
def _sc_kernels():
    mesh = plsc.VectorSubcoreMesh(core_axis_name="c", subcore_axis_name="s")
    common_scratch = [
        pltpu.VMEM((TPW,), jnp.int32),
        pltpu.VMEM((TPW,), jnp.int32),
        pltpu.VMEM((_NBUF, 16, HIDDEN), jnp.float32),
        pltpu.SemaphoreType.DMA,
        pltpu.SemaphoreType.DMA,
    ]
    dispatch = pl.kernel(
        _dispatch_sc_body,
        out_type=jax.ShapeDtypeStruct((S_PAD, HIDDEN), jnp.float32),
        mesh=mesh,
        scratch_types=common_scratch,
    )
    gather = pl.kernel(
        _gather_sc_body,
        out_type=(jax.ShapeDtypeStruct((N_TOKENS, HIDDEN), jnp.float32),
                  jax.ShapeDtypeStruct((N_TOKENS, HIDDEN), jnp.float32)),
        mesh=mesh,
        scratch_types=common_scratch,
    )
    return dispatch, gather


def _comb_body(p_ref, g0_ref, g1_ref, out_ref):
    p0 = p_ref[:, 0:1]
    p1 = p_ref[:, 1:2]
    out_ref[...] = p0 * g0_ref[...] + p1 * g1_ref[...]


def _combine(p01, G0, G1):
    tb = 1024
    return pl.pallas_call(
        _comb_body,
        grid=(N_TOKENS // tb,),
        in_specs=[
            pl.BlockSpec((tb, TOP_K), lambda i: (i, 0)),
            pl.BlockSpec((tb, HIDDEN), lambda i: (i, 0)),
            pl.BlockSpec((tb, HIDDEN), lambda i: (i, 0)),
        ],
        out_specs=pl.BlockSpec((tb, HIDDEN), lambda i: (i, 0)),
        out_shape=jax.ShapeDtypeStruct((N_TOKENS, HIDDEN), jnp.float32),
    )(p01, G0, G1)


@jax.jit
def kernel(tokens, Wr, W1, b1, W2, b2):
    slot01, p01, be2 = _route_meta(tokens, Wr)
    slot0 = slot01[:, 0]
    slot1 = slot01[:, 1]
    be = be2.reshape(NBLK)
    dispatch_k, gather_k = _sc_kernels()
    disp = dispatch_k(tokens, slot0, slot1)
    Y = _gmm(be, disp, W1, b1, W2, b2)
    G0, G1 = gather_k(Y, slot0, slot1)
    return _combine(p01, G0, G1)


# speedup vs baseline: 1.4640x; 1.4640x over previous
import functools

import jax
import jax.numpy as jnp
from jax import lax
from jax.experimental import pallas as pl
from jax.experimental.pallas import tpu as pltpu
from jax.experimental.pallas import tpu_sc as plsc

HIDDEN = 1024
FF = 2 * HIDDEN
NUM_EXPERTS = 8
N_TOKENS = 4096
TOP_K = 2
BLK = 512
NBLK = 24
S_PAD = NBLK * BLK
CHUNK = 512

NW = 32
TPW = N_TOKENS // NW


def _gelu_exact(x):
    return x * 0.5 * (1.0 + lax.erf(x * 0.7071067811865476))


def _meta_body(tokens_ref, wr_ref, slot_ref, p_ref, be_ref):
    x = tokens_ref[...]
    logits = lax.dot_general(x, wr_ref[...], (((1,), (1,)), ((), ())),
                             preferred_element_type=jnp.float32)
    idx8 = lax.broadcasted_iota(jnp.int32, logits.shape, 1)
    m1 = jnp.max(logits, axis=-1, keepdims=True)
    a1 = jnp.min(jnp.where(logits == m1, idx8, NUM_EXPERTS), axis=-1,
                 keepdims=True)
    l2 = jnp.where(idx8 == a1, jnp.float32(-1e30), logits)
    m2 = jnp.max(l2, axis=-1, keepdims=True)
    a2 = jnp.min(jnp.where(l2 == m2, idx8, NUM_EXPERTS), axis=-1,
                 keepdims=True)
    p1 = 1.0 / (1.0 + jnp.exp(m2 - m1))
    p2 = 1.0 - p1

    oh = (idx8 == a1).astype(jnp.float32) + (idx8 == a2).astype(jnp.float32)
    run = jnp.zeros((1, NUM_EXPERTS), jnp.float32)
    cum_chunks = []
    ri = lax.broadcasted_iota(jnp.int32, (CHUNK, CHUNK), 0)
    ci = lax.broadcasted_iota(jnp.int32, (CHUNK, CHUNK), 1)
    lstrict = (ci < ri).astype(jnp.float32)
    for c in range(N_TOKENS // CHUNK):
        blk = oh[c * CHUNK:(c + 1) * CHUNK]
        cum_chunks.append(
            lax.dot_general(lstrict, blk, (((1,), (0,)), ((), ())),
                            preferred_element_type=jnp.float32) + run)
        run = run + jnp.sum(blk, axis=0, keepdims=True)
    cum = jnp.concatenate(cum_chunks, axis=0)
    counts = run

    nblk_e = jnp.ceil(counts * (1.0 / BLK))
    er = lax.broadcasted_iota(jnp.int32, (NUM_EXPERTS, NUM_EXPERTS), 0)
    ec = lax.broadcasted_iota(jnp.int32, (NUM_EXPERTS, NUM_EXPERTS), 1)
    estrict = (er < ec).astype(jnp.float32)
    base_blk = lax.dot_general(nblk_e, estrict, (((1,), (0,)), ((), ())),
                               preferred_element_type=jnp.float32)
    base = base_blk * float(BLK)

    base_bc = jnp.broadcast_to(base, (N_TOKENS, NUM_EXPERTS))
    slot0 = jnp.sum(jnp.where(idx8 == a1, base_bc + cum, 0.0), axis=-1,
                    keepdims=True)
    slot1 = jnp.sum(jnp.where(idx8 == a2, base_bc + cum, 0.0), axis=-1,
                    keepdims=True)
    slot_ref[...] = jnp.concatenate([slot0, slot1], axis=1).astype(jnp.int32)
    p_ref[...] = jnp.concatenate([p1, p2], axis=1)

    bi = lax.broadcasted_iota(jnp.int32, (NBLK, NUM_EXPERTS), 0).astype(
        jnp.float32)
    cond = (jnp.broadcast_to(base_blk, (NBLK, NUM_EXPERTS)) <= bi)
    be_ref[...] = (jnp.sum(cond.astype(jnp.float32), axis=-1, keepdims=True)
                   - 1.0).astype(jnp.int32)


def _route_meta(tokens, Wr):
    return pl.pallas_call(
        _meta_body,
        in_specs=[
            pl.BlockSpec((N_TOKENS, HIDDEN), lambda: (0, 0)),
            pl.BlockSpec((NUM_EXPERTS, HIDDEN), lambda: (0, 0)),
        ],
        out_specs=[
            pl.BlockSpec((N_TOKENS, TOP_K), lambda: (0, 0)),
            pl.BlockSpec((N_TOKENS, TOP_K), lambda: (0, 0)),
            pl.BlockSpec((NBLK, 1), lambda: (0, 0)),
        ],
        out_shape=[
            jax.ShapeDtypeStruct((N_TOKENS, TOP_K), jnp.int32),
            jax.ShapeDtypeStruct((N_TOKENS, TOP_K), jnp.float32),
            jax.ShapeDtypeStruct((NBLK, 1), jnp.int32),
        ],
    )(tokens, Wr)


_NBUF = 3


def _dispatch_sc_body(tokens_hbm, slot0_hbm, slot1_hbm, disp_hbm,
                      idx0_v, idx1_v, xbuf, semin, semout):
    wid = lax.axis_index("s") * 2 + lax.axis_index("c")
    base = wid * TPW
    nch = TPW // 16
    pltpu.sync_copy(slot0_hbm.at[pl.ds(base, TPW)], idx0_v)
    pltpu.sync_copy(slot1_hbm.at[pl.ds(base, TPW)], idx1_v)
    stages = {}
    for ch in range(_NBUF):
        stages[ch] = pltpu.async_copy(
            tokens_hbm.at[pl.ds(base + ch * 16, 16)], xbuf.at[ch], semin)
    scats = []
    for ch in range(nch):
        b = ch % _NBUF
        stages[ch].wait()
        i0 = idx0_v[pl.ds(ch * 16, 16)]
        scats.append(pltpu.async_copy(xbuf.at[b], disp_hbm.at[i0], semout))
        i1 = idx1_v[pl.ds(ch * 16, 16)]
        scats.append(pltpu.async_copy(xbuf.at[b], disp_hbm.at[i1], semout))
        nxt = ch + _NBUF
        if nxt < nch:
            scats[2 * ch].wait()
            scats[2 * ch + 1].wait()
            stages[nxt] = pltpu.async_copy(
                tokens_hbm.at[pl.ds(base + nxt * 16, 16)], xbuf.at[b], semin)
    for u in range(2 * (nch - _NBUF), 2 * nch):
        scats[u].wait()


def _gmm_body(be_ref, x_ref, w1_ref, b1_ref, w2_ref, b2_ref, y_ref):
    del be_ref
    h = lax.dot_general(x_ref[...], w1_ref[0], (((1,), (1,)), ((), ())),
                        preferred_element_type=jnp.float32)
    h = _gelu_exact(h + b1_ref[0])
    y_ref[...] = lax.dot_general(h, w2_ref[0], (((1,), (1,)), ((), ())),
                                 preferred_element_type=jnp.float32) + b2_ref[0]


def _gmm(be, disp, W1, b1, W2, b2):
    grid_spec = pltpu.PrefetchScalarGridSpec(
        num_scalar_prefetch=1,
        grid=(NBLK,),
        in_specs=[
            pl.BlockSpec((BLK, HIDDEN), lambda i, be: (i, 0)),
            pl.BlockSpec((1, FF, HIDDEN), lambda i, be: (be[i], 0, 0)),
            pl.BlockSpec((1, 1, FF), lambda i, be: (be[i], 0, 0)),
            pl.BlockSpec((1, HIDDEN, FF), lambda i, be: (be[i], 0, 0)),
            pl.BlockSpec((1, 1, HIDDEN), lambda i, be: (be[i], 0, 0)),
        ],
        out_specs=pl.BlockSpec((BLK, HIDDEN), lambda i, be: (i, 0)),
    )
    return pl.pallas_call(
        _gmm_body,
        grid_spec=grid_spec,
        out_shape=jax.ShapeDtypeStruct((S_PAD, HIDDEN), jnp.float32),
    )(be, disp, W1, b1[:, None, :], W2, b2[:, None, :])


def _gather_sc_body(y_hbm, slot0_hbm, slot1_hbm, g0_hbm, g1_hbm,
                    idx0_v, idx1_v, rbuf, semin, semout):
    wid = lax.axis_index("s") * 2 + lax.axis_index("c")
    base = wid * TPW
    nch = TPW // 16
    pltpu.sync_copy(slot0_hbm.at[pl.ds(base, TPW)], idx0_v)
    pltpu.sync_copy(slot1_hbm.at[pl.ds(base, TPW)], idx1_v)

    def _idx(u):
        k, ch = u % 2, u // 2
        iv = idx0_v if k == 0 else idx1_v
        return iv[pl.ds(ch * 16, 16)]

    def _dst(u):
        k, ch = u % 2, u // 2
        g = g0_hbm if k == 0 else g1_hbm
        return g.at[pl.ds(base + ch * 16, 16)]

    nun = 2 * nch
    gath = {}
    for u in range(_NBUF):
        gath[u] = pltpu.async_copy(y_hbm.at[_idx(u)], rbuf.at[u], semin)
    outs = []
    for u in range(nun):
        b = u % _NBUF
        gath[u].wait()
        outs.append(pltpu.async_copy(rbuf.at[b], _dst(u), semout))
        nxt = u + _NBUF
        if nxt < nun:
            outs[u].wait()
            gath[nxt] = pltpu.async_copy(y_hbm.at[_idx(nxt)], rbuf.at[b],
                                         semin)
    for u in range(nun - _NBUF, nun):
        outs[u].wait()


@functools.lru_cache(maxsize=None)
def _sc_kernels():
    mesh = plsc.VectorSubcoreMesh(core_axis_name="c", subcore_axis_name="s")
    common_scratch = [
        pltpu.VMEM((TPW,), jnp.int32),
        pltpu.VMEM((TPW,), jnp.int32),
        pltpu.VMEM((_NBUF, 16, HIDDEN), jnp.float32),
        pltpu.SemaphoreType.DMA,
        pltpu.SemaphoreType.DMA,
    ]
    dispatch = pl.kernel(
        _dispatch_sc_body,
        out_type=jax.ShapeDtypeStruct((S_PAD, HIDDEN), jnp.float32),
        mesh=mesh,
        scratch_types=common_scratch,
    )
    gather = pl.kernel(
        _gather_sc_body,
        out_type=(jax.ShapeDtypeStruct((N_TOKENS, HIDDEN), jnp.float32),
                  jax.ShapeDtypeStruct((N_TOKENS, HIDDEN), jnp.float32)),
        mesh=mesh,
        scratch_types=common_scratch,
    )
    return dispatch, gather


def _comb_body(p_ref, g0_ref, g1_ref, out_ref):
    p0 = p_ref[:, 0:1]
    p1 = p_ref[:, 1:2]
    out_ref[...] = p0 * g0_ref[...] + p1 * g1_ref[...]


def _combine(p01, G0, G1):
    tb = 1024
    return pl.pallas_call(
        _comb_body,
        grid=(N_TOKENS // tb,),
        in_specs=[
            pl.BlockSpec((tb, TOP_K), lambda i: (i, 0)),
            pl.BlockSpec((tb, HIDDEN), lambda i: (i, 0)),
            pl.BlockSpec((tb, HIDDEN), lambda i: (i, 0)),
        ],
        out_specs=pl.BlockSpec((tb, HIDDEN), lambda i: (i, 0)),
        out_shape=jax.ShapeDtypeStruct((N_TOKENS, HIDDEN), jnp.float32),
    )(p01, G0, G1)


@jax.jit
def kernel(tokens, Wr, W1, b1, W2, b2):
    slot01, p01, be2 = _route_meta(tokens, Wr)
    slot0 = slot01[:, 0]
    slot1 = slot01[:, 1]
    be = be2.reshape(NBLK)
    dispatch_k, gather_k = _sc_kernels()
    disp = dispatch_k(tokens, slot0, slot1)
    Y = _gmm(be, disp, W1, b1, W2, b2)
    G0, G1 = gather_k(Y, slot0, slot1)
    return _combine(p01, G0, G1)


# speedup vs baseline: 1.5517x; 1.0599x over previous
import functools

import jax
import jax.numpy as jnp
from jax import lax
from jax.experimental import pallas as pl
from jax.experimental.pallas import tpu as pltpu
from jax.experimental.pallas import tpu_sc as plsc

HIDDEN = 1024
FF = 2 * HIDDEN
NUM_EXPERTS = 8
N_TOKENS = 4096
TOP_K = 2
BLK = 512
NBLK = 24
S_PAD = NBLK * BLK
CHUNK = 512

NW = 32
TPW = N_TOKENS // NW


def _gelu_exact(x):
    return x * 0.5 * (1.0 + lax.erf(x * 0.7071067811865476))


def _meta_body(tokens_ref, wr_ref, slot_ref, p_ref, be_ref):
    x = tokens_ref[...]
    logits = lax.dot_general(x, wr_ref[...], (((1,), (1,)), ((), ())),
                             preferred_element_type=jnp.float32)
    idx8 = lax.broadcasted_iota(jnp.int32, logits.shape, 1)
    m1 = jnp.max(logits, axis=-1, keepdims=True)
    a1 = jnp.min(jnp.where(logits == m1, idx8, NUM_EXPERTS), axis=-1,
                 keepdims=True)
    l2 = jnp.where(idx8 == a1, jnp.float32(-1e30), logits)
    m2 = jnp.max(l2, axis=-1, keepdims=True)
    a2 = jnp.min(jnp.where(l2 == m2, idx8, NUM_EXPERTS), axis=-1,
                 keepdims=True)
    p1 = 1.0 / (1.0 + jnp.exp(m2 - m1))
    p2 = 1.0 - p1

    oh = (idx8 == a1).astype(jnp.float32) + (idx8 == a2).astype(jnp.float32)
    run = jnp.zeros((1, NUM_EXPERTS), jnp.float32)
    cum_chunks = []
    ri = lax.broadcasted_iota(jnp.int32, (CHUNK, CHUNK), 0)
    ci = lax.broadcasted_iota(jnp.int32, (CHUNK, CHUNK), 1)
    lstrict = (ci < ri).astype(jnp.float32)
    for c in range(N_TOKENS // CHUNK):
        blk = oh[c * CHUNK:(c + 1) * CHUNK]
        cum_chunks.append(
            lax.dot_general(lstrict, blk, (((1,), (0,)), ((), ())),
                            preferred_element_type=jnp.float32) + run)
        run = run + jnp.sum(blk, axis=0, keepdims=True)
    cum = jnp.concatenate(cum_chunks, axis=0)
    counts = run

    nblk_e = jnp.ceil(counts * (1.0 / BLK))
    er = lax.broadcasted_iota(jnp.int32, (NUM_EXPERTS, NUM_EXPERTS), 0)
    ec = lax.broadcasted_iota(jnp.int32, (NUM_EXPERTS, NUM_EXPERTS), 1)
    estrict = (er < ec).astype(jnp.float32)
    base_blk = lax.dot_general(nblk_e, estrict, (((1,), (0,)), ((), ())),
                               preferred_element_type=jnp.float32)
    base = base_blk * float(BLK)

    base_bc = jnp.broadcast_to(base, (N_TOKENS, NUM_EXPERTS))
    slot0 = jnp.sum(jnp.where(idx8 == a1, base_bc + cum, 0.0), axis=-1,
                    keepdims=True)
    slot1 = jnp.sum(jnp.where(idx8 == a2, base_bc + cum, 0.0), axis=-1,
                    keepdims=True)
    slot_ref[...] = jnp.concatenate([slot0, slot1], axis=1).astype(jnp.int32)
    p_ref[...] = jnp.concatenate([p1, p2], axis=1)

    bi = lax.broadcasted_iota(jnp.int32, (NBLK, NUM_EXPERTS), 0).astype(
        jnp.float32)
    cond = (jnp.broadcast_to(base_blk, (NBLK, NUM_EXPERTS)) <= bi)
    be_ref[...] = (jnp.sum(cond.astype(jnp.float32), axis=-1, keepdims=True)
                   - 1.0).astype(jnp.int32)


def _route_meta(tokens, Wr):
    return pl.pallas_call(
        _meta_body,
        in_specs=[
            pl.BlockSpec((N_TOKENS, HIDDEN), lambda: (0, 0)),
            pl.BlockSpec((NUM_EXPERTS, HIDDEN), lambda: (0, 0)),
        ],
        out_specs=[
            pl.BlockSpec((N_TOKENS, TOP_K), lambda: (0, 0)),
            pl.BlockSpec((N_TOKENS, TOP_K), lambda: (0, 0)),
            pl.BlockSpec((NBLK, 1), lambda: (0, 0)),
        ],
        out_shape=[
            jax.ShapeDtypeStruct((N_TOKENS, TOP_K), jnp.int32),
            jax.ShapeDtypeStruct((N_TOKENS, TOP_K), jnp.float32),
            jax.ShapeDtypeStruct((NBLK, 1), jnp.int32),
        ],
    )(tokens, Wr)


_NBUF = 3


def _dispatch_sc_body(tokens_hbm, slot0_hbm, slot1_hbm, disp_hbm,
                      idx0_v, idx1_v, xbuf, semin, semout):
    wid = lax.axis_index("s") * 2 + lax.axis_index("c")
    base = wid * TPW
    nch = TPW // 16
    pltpu.sync_copy(slot0_hbm.at[pl.ds(base, TPW)], idx0_v)
    pltpu.sync_copy(slot1_hbm.at[pl.ds(base, TPW)], idx1_v)
    stages = {}
    for ch in range(_NBUF):
        stages[ch] = pltpu.async_copy(
            tokens_hbm.at[pl.ds(base + ch * 16, 16)], xbuf.at[ch], semin)
    scats = []
    for ch in range(nch):
        b = ch % _NBUF
        stages[ch].wait()
        i0 = idx0_v[pl.ds(ch * 16, 16)]
        scats.append(pltpu.async_copy(xbuf.at[b], disp_hbm.at[i0], semout))
        i1 = idx1_v[pl.ds(ch * 16, 16)]
        scats.append(pltpu.async_copy(xbuf.at[b], disp_hbm.at[i1], semout))
        nxt = ch + _NBUF
        if nxt < nch:
            scats[2 * ch].wait()
            scats[2 * ch + 1].wait()
            stages[nxt] = pltpu.async_copy(
                tokens_hbm.at[pl.ds(base + nxt * 16, 16)], xbuf.at[b], semin)
    for u in range(2 * (nch - _NBUF), 2 * nch):
        scats[u].wait()


def _gmm_body(be_ref, x_ref, w1_ref, b1_ref, w2_ref, b2_ref, y_ref):
    del be_ref
    h = lax.dot_general(x_ref[...], w1_ref[0], (((1,), (1,)), ((), ())),
                        preferred_element_type=jnp.float32)
    h = _gelu_exact(h + b1_ref[0])
    y_ref[...] = lax.dot_general(h, w2_ref[0], (((1,), (1,)), ((), ())),
                                 preferred_element_type=jnp.float32) + b2_ref[0]


def _gmm(be, disp, W1, b1, W2, b2):
    grid_spec = pltpu.PrefetchScalarGridSpec(
        num_scalar_prefetch=1,
        grid=(NBLK,),
        in_specs=[
            pl.BlockSpec((BLK, HIDDEN), lambda i, be: (i, 0)),
            pl.BlockSpec((1, FF, HIDDEN), lambda i, be: (be[i], 0, 0)),
            pl.BlockSpec((1, 1, FF), lambda i, be: (be[i], 0, 0)),
            pl.BlockSpec((1, HIDDEN, FF), lambda i, be: (be[i], 0, 0)),
            pl.BlockSpec((1, 1, HIDDEN), lambda i, be: (be[i], 0, 0)),
        ],
        out_specs=pl.BlockSpec((BLK, HIDDEN), lambda i, be: (i, 0)),
    )
    return pl.pallas_call(
        _gmm_body,
        grid_spec=grid_spec,
        out_shape=jax.ShapeDtypeStruct((S_PAD, HIDDEN), jnp.float32),
    )(be, disp, W1, b1[:, None, :], W2, b2[:, None, :])


_SPLAT_DNUMS = lax.GatherDimensionNumbers(
    offset_dims=(), collapsed_slice_dims=(0,), start_index_map=(0,))


def _splat(vec, j):
    idx = jnp.full((16, 1), j, jnp.int32)
    return lax.gather(vec, idx, _SPLAT_DNUMS, (1,),
                      mode=lax.GatherScatterMode.PROMISE_IN_BOUNDS)


def _gather_sc_body(y_hbm, slot0_hbm, slot1_hbm, p0_hbm, p1_hbm, out_hbm,
                    idx0_v, idx1_v, p0_v, p1_v, r0, r1, obuf,
                    semg0, semg1, semout):
    wid = lax.axis_index("s") * 2 + lax.axis_index("c")
    base = wid * TPW
    nch = TPW // 16
    pltpu.sync_copy(slot0_hbm.at[pl.ds(base, TPW)], idx0_v)
    pltpu.sync_copy(slot1_hbm.at[pl.ds(base, TPW)], idx1_v)
    pltpu.sync_copy(p0_hbm.at[pl.ds(base, TPW)], p0_v)
    pltpu.sync_copy(p1_hbm.at[pl.ds(base, TPW)], p1_v)

    def _issue(ch):
        b = ch % 2
        sem = semg0 if b == 0 else semg1
        h0 = pltpu.async_copy(y_hbm.at[idx0_v[pl.ds(ch * 16, 16)]],
                              r0.at[b], sem)
        h1 = pltpu.async_copy(y_hbm.at[idx1_v[pl.ds(ch * 16, 16)]],
                              r1.at[b], sem)
        return h0, h1

    gath = {0: _issue(0)}
    outs = []
    for ch in range(nch):
        b = ch % 2
        if ch + 1 < nch:
            gath[ch + 1] = _issue(ch + 1)
        gath[ch][0].wait()
        gath[ch][1].wait()
        if ch >= 2:
            outs[ch - 2].wait()
        p0ch = p0_v[pl.ds(ch * 16, 16)]
        p1ch = p1_v[pl.ds(ch * 16, 16)]
        g0s = [_splat(p0ch, j) for j in range(16)]
        g1s = [_splat(p1ch, j) for j in range(16)]

        def _cbody(c, carry):
            for j in range(16):
                sl = pl.ds(c * 16, 16)
                obuf[b, j, sl] = (g0s[j] * r0[b, j, sl]
                                  + g1s[j] * r1[b, j, sl])
            return carry

        lax.fori_loop(0, HIDDEN // 16, _cbody, 0)
        outs.append(pltpu.async_copy(
            obuf.at[b], out_hbm.at[pl.ds(base + ch * 16, 16)], semout))
    outs[nch - 2].wait()
    outs[nch - 1].wait()


@functools.lru_cache(maxsize=None)
def _sc_kernels():
    mesh = plsc.VectorSubcoreMesh(core_axis_name="c", subcore_axis_name="s")
    common_scratch = [
        pltpu.VMEM((TPW,), jnp.int32),
        pltpu.VMEM((TPW,), jnp.int32),
        pltpu.VMEM((_NBUF, 16, HIDDEN), jnp.float32),
        pltpu.SemaphoreType.DMA,
        pltpu.SemaphoreType.DMA,
    ]
    dispatch = pl.kernel(
        _dispatch_sc_body,
        out_type=jax.ShapeDtypeStruct((S_PAD, HIDDEN), jnp.float32),
        mesh=mesh,
        scratch_types=common_scratch,
    )
    gather = pl.kernel(
        _gather_sc_body,
        out_type=jax.ShapeDtypeStruct((N_TOKENS, HIDDEN), jnp.float32),
        mesh=mesh,
        scratch_types=[
            pltpu.VMEM((TPW,), jnp.int32),
            pltpu.VMEM((TPW,), jnp.int32),
            pltpu.VMEM((TPW,), jnp.float32),
            pltpu.VMEM((TPW,), jnp.float32),
            pltpu.VMEM((2, 16, HIDDEN), jnp.float32),
            pltpu.VMEM((2, 16, HIDDEN), jnp.float32),
            pltpu.VMEM((2, 16, HIDDEN), jnp.float32),
            pltpu.SemaphoreType.DMA,
            pltpu.SemaphoreType.DMA,
            pltpu.SemaphoreType.DMA,
        ],
    )
    return dispatch, gather


def _comb_body(p_ref, g0_ref, g1_ref, out_ref):
    p0 = p_ref[:, 0:1]
    p1 = p_ref[:, 1:2]
    out_ref[...] = p0 * g0_ref[...] + p1 * g1_ref[...]


def _combine(p01, G0, G1):
    tb = 1024
    return pl.pallas_call(
        _comb_body,
        grid=(N_TOKENS // tb,),
        in_specs=[
            pl.BlockSpec((tb, TOP_K), lambda i: (i, 0)),
            pl.BlockSpec((tb, HIDDEN), lambda i: (i, 0)),
            pl.BlockSpec((tb, HIDDEN), lambda i: (i, 0)),
        ],
        out_specs=pl.BlockSpec((tb, HIDDEN), lambda i: (i, 0)),
        out_shape=jax.ShapeDtypeStruct((N_TOKENS, HIDDEN), jnp.float32),
    )(p01, G0, G1)


@jax.jit
def kernel(tokens, Wr, W1, b1, W2, b2):
    slot01, p01, be2 = _route_meta(tokens, Wr)
    slot0 = slot01[:, 0]
    slot1 = slot01[:, 1]
    be = be2.reshape(NBLK)
    dispatch_k, gather_k = _sc_kernels()
    disp = dispatch_k(tokens, slot0, slot1)
    Y = _gmm(be, disp, W1, b1, W2, b2)
    return gather_k(Y, slot0, slot1, p01[:, 0], p01[:, 1])
